# baseline (speedup 1.0000x reference)
import jax
import jax.numpy as jnp
from jax import lax
from jax.experimental import pallas as pl
from jax.experimental.pallas import tpu as pltpu
from jax.experimental.pallas import tpu_sc as plsc

B, N, NCOUT = 8, 2048, 128
KNB = 15
KPAD = 16
PW = 16
RB = 512
PB = 512
NEG = -1e30

E = B * N * KPAD
SC_TILES = 32
IDX_ROWS = E // 128
ROWS_PER_TILE = IDX_ROWS // SC_TILES


def _select_body(x_ref, xt_ref, idx_ref):
    b = pl.program_id(0)
    i = pl.program_id(1)
    xb = x_ref[0]
    xt = xt_ref[0]
    xbb = xb.astype(jnp.bfloat16).astype(jnp.float32)
    xtb = xt.astype(jnp.bfloat16).astype(jnp.float32)
    p0 = xbb[:, 0:1] * xtb[0:1, :]
    p1 = xbb[:, 1:2] * xtb[1:2, :]
    p2 = xbb[:, 2:3] * xtb[2:3, :]
    s1 = p0 + p1
    ap = s1 - p1
    e1 = (p0 - ap) + (p1 - (s1 - ap))
    s2 = s1 + p2
    ap2 = s2 - p2
    e2 = (s1 - ap2) + (p2 - (s2 - ap2))
    inner = s2 + (e1 + e2)
    xb0, xb1, xb2 = xb[:, 0:1], xb[:, 1:2], xb[:, 2:3]
    xt0, xt1, xt2 = xt[0:1, :], xt[1:2, :], xt[2:3, :]
    xx_blk = xb0 * xb0 + xb1 * xb1 + xb2 * xb2
    xx_all = xt0 * xt0 + xt1 * xt1 + xt2 * xt2
    nd = 2.0 * inner - xx_blk - xx_all
    slot = lax.broadcasted_iota(jnp.int32, (RB, KPAD), 1)
    acc0 = i * RB + lax.broadcasted_iota(jnp.int32, (RB, KPAD), 0)
    a0 = jnp.full((RB, 1), -1, jnp.int32)
    col = lax.broadcasted_iota(jnp.int32, (RB, N), 1)
    BIGI = jnp.int32(1 << 30)

    def step(j, carry):
        nd, a1p, a2p, acc = carry
        nd = jnp.where((col == a1p) | (col == a2p), NEG, nd)
        m1 = jnp.max(nd, axis=1, keepdims=True)
        hit1 = nd == m1
        cols1 = jnp.where(hit1, col, BIGI)
        a1 = jnp.min(cols1, axis=1, keepdims=True)
        a2dup = jnp.min(jnp.where(cols1 > a1, cols1, BIGI),
                        axis=1, keepdims=True)
        m2 = jnp.max(jnp.where(hit1, NEG, nd), axis=1, keepdims=True)
        a2m2 = jnp.min(jnp.where(nd == m2, col, BIGI), axis=1, keepdims=True)
        a2 = jnp.where(a2dup < BIGI, a2dup, a2m2)
        acc = jnp.where(slot == 2 * j - 1, a1, acc)
        acc = jnp.where(slot == 2 * j, a2, acc)
        return nd, a1, a2, acc

    _, _, _, acc = lax.fori_loop(0, KPAD // 2, step, (nd, a0, a0, acc0))
    idx_ref[...] = acc + b * N


def _build_select(interpret=False):
    return pl.pallas_call(
        _select_body,
        grid=(B, N // RB),
        in_specs=[
            pl.BlockSpec((1, RB, PW), lambda b, i: (b, i, 0)),
            pl.BlockSpec((1, PW, N), lambda b, i: (b, 0, 0)),
        ],
        out_specs=pl.BlockSpec((RB, KPAD), lambda b, i: (b * (N // RB) + i, 0)),
        out_shape=jax.ShapeDtypeStruct((B * N, KPAD), jnp.int32),
        interpret=interpret,
    )


def _gather_sc_body(table_hbm, idx_hbm, out_hbm, idx_v, rows_a, sem_g):
    c = lax.axis_index("c")
    s = lax.axis_index("s")
    wid = s * 2 + c
    pltpu.sync_copy(idx_hbm.at[pl.ds(wid * ROWS_PER_TILE, ROWS_PER_TILE)], idx_v)
    base = wid * ROWS_PER_TILE * 128

    def body(g, _):
        cp = pltpu.async_copy(table_hbm.at[idx_v.at[g]], rows_a, sem_g)
        cp.wait()
        pltpu.sync_copy(rows_a, out_hbm.at[pl.ds(base + g * 128, 128)])
        return 0

    lax.fori_loop(0, ROWS_PER_TILE, body, 0)


def _build_gather():
    mesh = plsc.VectorSubcoreMesh(core_axis_name="c", subcore_axis_name="s")
    return pl.kernel(
        _gather_sc_body,
        out_type=jax.ShapeDtypeStruct((E, PW), jnp.float32),
        mesh=mesh,
        scratch_types=[
            pltpu.VMEM((ROWS_PER_TILE, 128), jnp.int32),
            pltpu.VMEM((128, PW), jnp.float32),
            pltpu.SemaphoreType.DMA,
        ],
        compiler_params=pltpu.CompilerParams(use_tc_tiling_on_sc=False),
    )


def _mlp_body(g_ref, xc_ref, w1_ref, w2_ref, w3_ref, out_ref):
    xc = xc_ref[...]
    gw = g_ref[...]
    d = jnp.concatenate(
        [gw[:, k * PW:(k + 1) * PW] - xc for k in range(KPAD)], axis=0)
    h = jnp.maximum(jnp.dot(d, w1_ref[...], preferred_element_type=jnp.float32), 0.0)
    h = jnp.maximum(jnp.dot(h, w2_ref[...], preferred_element_type=jnp.float32), 0.0)
    h = jnp.dot(h, w3_ref[...], preferred_element_type=jnp.float32)
    h = h.reshape(KPAD, PB, NCOUT)
    out_ref[...] = jnp.max(h[:KNB], axis=0)


def _build_mlp(interpret=False):
    return pl.pallas_call(
        _mlp_body,
        grid=(B * N // PB,),
        in_specs=[
            pl.BlockSpec((PB, KPAD * PW), lambda p: (p, 0)),
            pl.BlockSpec((PB, PW), lambda p: (p, 0)),
            pl.BlockSpec((PW, NCOUT), lambda p: (0, 0)),
            pl.BlockSpec((NCOUT, NCOUT), lambda p: (0, 0)),
            pl.BlockSpec((NCOUT, NCOUT), lambda p: (0, 0)),
        ],
        out_specs=pl.BlockSpec((PB, NCOUT), lambda p: (p, 0)),
        out_shape=jax.ShapeDtypeStruct((B * N, NCOUT), jnp.float32),
        interpret=interpret,
    )


def kernel(xyz, W1, W2, W3):
    x = lax.stop_gradient(xyz)
    x_pad = jnp.pad(x, ((0, 0), (0, 0), (0, PW - 3)))
    xt = jnp.transpose(x_pad, (0, 2, 1))
    idx = _build_select()(x_pad, xt)
    table = x_pad.reshape(B * N, PW)
    gathered = _build_gather()(table, idx.reshape(IDX_ROWS, 128))
    w1t = jnp.pad(W1.T, ((0, PW - 3), (0, 0)))
    out = _build_mlp()(
        gathered.reshape(B * N, KPAD * PW), table, w1t, W2.T, W3.T
    )
    return out.reshape(B, N, NCOUT).transpose(0, 2, 1)
